# baseline (speedup 1.0000x reference)
import functools

import jax
import jax.numpy as jnp
from jax import lax
from jax.experimental import pallas as pl
from jax.experimental.pallas import tpu as pltpu
from jax.experimental.pallas import tpu_sc as plsc

BATCH = 16384
HIST = 50
DIM = 128
NW = 32
CHUNK_B = 2
CHUNK = CHUNK_B * HIST
NBUF = 4
N_SLICES = 4
SLICE_B = BATCH // N_SLICES


def _make_kernel(n_batches):
    batch_per_w = n_batches // NW
    n_chunks = batch_per_w // CHUNK_B
    n_groups = n_chunks // NBUF
    mesh = plsc.VectorSubcoreMesh(core_axis_name="c", subcore_axis_name="s")

    @functools.partial(
        pl.kernel,
        out_type=jax.ShapeDtypeStruct((n_batches, HIST, DIM), jnp.float32),
        mesh=mesh,
        scratch_types=[
            pltpu.VMEM((n_chunks, CHUNK), jnp.int32),
            [pltpu.VMEM((CHUNK, DIM), jnp.float32) for _ in range(NBUF)],
            [pltpu.SemaphoreType.DMA for _ in range(NBUF)],
            [pltpu.SemaphoreType.DMA for _ in range(NBUF)],
        ],
    )
    def gather_kernel(idx_hbm, table_hbm, out_hbm, idx_v, rows, gsems, ssems):
        wid = lax.axis_index("s") * 2 + lax.axis_index("c")
        base_b = wid * batch_per_w
        pltpu.sync_copy(idx_hbm.at[wid], idx_v)

        def start_gather(b, j):
            pltpu.async_copy(table_hbm.at[idx_v.at[j]], rows[b], gsems[b])

        def wait_gather(b):
            pltpu.make_async_copy(table_hbm.at[idx_v.at[0]], rows[b], gsems[b]).wait()

        def start_store(b, j):
            b0 = base_b + j * CHUNK_B
            pltpu.async_copy(rows[b].at[pl.ds(0, HIST)], out_hbm.at[b0], ssems[b])
            pltpu.async_copy(
                rows[b].at[pl.ds(HIST, HIST)], out_hbm.at[b0 + 1], ssems[b]
            )

        def wait_store(b):
            pltpu.make_async_copy(
                rows[b].at[pl.ds(0, HIST)], out_hbm.at[0], ssems[b]
            ).wait()
            pltpu.make_async_copy(
                rows[b].at[pl.ds(0, HIST)], out_hbm.at[0], ssems[b]
            ).wait()

        for b in range(NBUF):
            start_gather(b, b)

        def body(g, carry):
            j0 = g * NBUF
            for b in range(NBUF):
                wait_gather(b)
                start_store(b, j0 + b)
            for b in range(NBUF):
                wait_store(b)
                start_gather(b, j0 + NBUF + b)
            return carry

        lax.fori_loop(0, n_groups - 1, body, 0)

        j0 = (n_groups - 1) * NBUF
        for b in range(NBUF):
            wait_gather(b)
            start_store(b, j0 + b)
        for b in range(NBUF):
            wait_store(b)

    return gather_kernel


_gather = _make_kernel(SLICE_B)


@jax.jit
def kernel(y_true, proxies):
    idx = y_true.astype(jnp.int32)
    outs = []
    for s in range(N_SLICES):
        sl = idx[s * SLICE_B:(s + 1) * SLICE_B]
        n_chunks = SLICE_B // NW // CHUNK_B
        outs.append(_gather(sl.reshape(NW, n_chunks, CHUNK), proxies))
    return jnp.concatenate(outs, axis=0)
